# baseline (speedup 1.0000x reference)
import functools

import jax
import jax.numpy as jnp
from jax import lax
from jax.experimental import pallas as pl
from jax.experimental.pallas import tpu as pltpu
from jax.experimental.pallas import tpu_sc as plsc

B = 16384
D = 128
H1, H2, H3 = 1024, 512, 128

NC, NS = 2, 16
NW = NC * NS
BPW = B // NW
CH = 128
NCH = BPW // CH


def _sc_gather(table, idx):
    mesh = plsc.VectorSubcoreMesh(core_axis_name="c", subcore_axis_name="s",
                                  num_cores=NC, num_subcores=NS)

    @functools.partial(
        pl.kernel,
        out_type=jax.ShapeDtypeStruct((B, D), jnp.float32),
        mesh=mesh,
        scratch_types=[
            pltpu.VMEM((NCH, CH), jnp.int32),
            pltpu.VMEM((BPW, D), jnp.float32),
            pltpu.SemaphoreType.DMA,
        ],
    )
    def gather_k(tab_hbm, idx_hbm, out_hbm, idx_v, rows_v, sem):
        wid = lax.axis_index("s") * NC + lax.axis_index("c")
        pltpu.sync_copy(idx_hbm.at[pl.ds(wid * NCH, NCH)], idx_v)
        cps = [
            pltpu.async_copy(tab_hbm.at[idx_v.at[j]],
                             rows_v.at[pl.ds(j * CH, CH)], sem)
            for j in range(NCH)
        ]
        for cp in cps:
            cp.wait()
        pltpu.sync_copy(rows_v, out_hbm.at[pl.ds(wid * BPW, BPW)])

    return gather_k(table, idx)


def _mlp_body(x_ref, w1_ref, b1_ref, w2_ref, b2_ref, w3_ref, b3_ref, o_ref):
    h = jnp.dot(x_ref[...], w1_ref[...], preferred_element_type=jnp.float32)
    h = jnp.maximum(h + b1_ref[...], 0.0)
    h = jnp.dot(h, w2_ref[...], preferred_element_type=jnp.float32)
    h = jnp.maximum(h + b2_ref[...], 0.0)
    o_ref[...] = jnp.dot(h, w3_ref[...],
                         preferred_element_type=jnp.float32) + b3_ref[...]


def _mlp_tc(x, w1, b1, w2, b2, w3, b3, bt=1024, interpret=False):
    nbt = B // bt
    return pl.pallas_call(
        _mlp_body,
        grid=(nbt,),
        in_specs=[
            pl.BlockSpec((bt, D), lambda b: (b, 0)),
            pl.BlockSpec((D, H1), lambda b: (0, 0)),
            pl.BlockSpec((1, H1), lambda b: (0, 0)),
            pl.BlockSpec((H1, H2), lambda b: (0, 0)),
            pl.BlockSpec((1, H2), lambda b: (0, 0)),
            pl.BlockSpec((H2, H3), lambda b: (0, 0)),
            pl.BlockSpec((1, H3), lambda b: (0, 0)),
        ],
        out_specs=pl.BlockSpec((bt, H3), lambda b: (b, 0)),
        out_shape=jax.ShapeDtypeStruct((B, H3), jnp.float32),
        interpret=interpret,
    )(x, w1, b1, w2, b2, w3, b3)


def kernel(user_input, item_input, user_table, item_table,
           uW1, ub1, uW2, ub2, uW3, ub3,
           iW1, ib1, iW2, ib2, iW3, ib3):
    uidx = user_input.astype(jnp.int32).reshape(NW * NCH, CH)
    iidx = item_input.astype(jnp.int32).reshape(NW * NCH, CH)
    user_rows = _sc_gather(user_table, uidx)
    item_rows = _sc_gather(item_table, iidx)
    user_embedding = _mlp_tc(user_rows, uW1, ub1.reshape(1, H1),
                             uW2, ub2.reshape(1, H2), uW3, ub3.reshape(1, H3))
    item_embedding = _mlp_tc(item_rows, iW1, ib1.reshape(1, H1),
                             iW2, ib2.reshape(1, H2), iW3, ib3.reshape(1, H3))
    return user_embedding, item_embedding


# speedup vs baseline: 1.0559x; 1.0559x over previous
import functools

import jax
import jax.numpy as jnp
from jax import lax
from jax.experimental import pallas as pl
from jax.experimental.pallas import tpu as pltpu
from jax.experimental.pallas import tpu_sc as plsc

B = 16384
D = 128
H1, H2, H3 = 1024, 512, 128

NC, NS = 2, 16
NW = NC * NS
BPW = B // NW
CH = 128
NCH = BPW // CH


def _sc_gather(table, idx):
    mesh = plsc.VectorSubcoreMesh(core_axis_name="c", subcore_axis_name="s",
                                  num_cores=NC, num_subcores=NS)

    @functools.partial(
        pl.kernel,
        out_type=jax.ShapeDtypeStruct((B, D), jnp.float32),
        mesh=mesh,
        scratch_types=[
            pltpu.VMEM((NCH, CH), jnp.int32),
            pltpu.VMEM((BPW, D), jnp.float32),
            pltpu.SemaphoreType.DMA,
        ],
    )
    def gather_k(tab_hbm, idx_hbm, out_hbm, idx_v, rows_v, sem):
        wid = lax.axis_index("s") * NC + lax.axis_index("c")
        pltpu.sync_copy(idx_hbm.at[pl.ds(wid * NCH, NCH)], idx_v)
        cps = [
            pltpu.async_copy(tab_hbm.at[idx_v.at[j]],
                             rows_v.at[pl.ds(j * CH, CH)], sem)
            for j in range(NCH)
        ]
        for cp in cps:
            cp.wait()
        pltpu.sync_copy(rows_v, out_hbm.at[pl.ds(wid * BPW, BPW)])

    return gather_k(table, idx)


def _mlp_body(x_ref, w1_ref, b1_ref, w2_ref, b2_ref, w3_ref, b3_ref, o_ref):
    h = jnp.dot(x_ref[...], w1_ref[...], preferred_element_type=jnp.float32)
    h = jnp.maximum(h + b1_ref[...], 0.0)
    h = jnp.dot(h, w2_ref[...], preferred_element_type=jnp.float32)
    h = jnp.maximum(h + b2_ref[...], 0.0)
    o_ref[...] = jnp.dot(h, w3_ref[...],
                         preferred_element_type=jnp.float32) + b3_ref[...]


def _mlp_tc(x, w1, b1, w2, b2, w3, b3, bt=4096, interpret=False):
    nbt = B // bt
    return pl.pallas_call(
        _mlp_body,
        grid=(nbt,),
        in_specs=[
            pl.BlockSpec((bt, D), lambda b: (b, 0)),
            pl.BlockSpec((D, H1), lambda b: (0, 0)),
            pl.BlockSpec((1, H1), lambda b: (0, 0)),
            pl.BlockSpec((H1, H2), lambda b: (0, 0)),
            pl.BlockSpec((1, H2), lambda b: (0, 0)),
            pl.BlockSpec((H2, H3), lambda b: (0, 0)),
            pl.BlockSpec((1, H3), lambda b: (0, 0)),
        ],
        out_specs=pl.BlockSpec((bt, H3), lambda b: (b, 0)),
        out_shape=jax.ShapeDtypeStruct((B, H3), jnp.float32),
        interpret=interpret,
    )(x, w1, b1, w2, b2, w3, b3)


def kernel(user_input, item_input, user_table, item_table,
           uW1, ub1, uW2, ub2, uW3, ub3,
           iW1, ib1, iW2, ib2, iW3, ib3):
    uidx = user_input.astype(jnp.int32).reshape(NW * NCH, CH)
    iidx = item_input.astype(jnp.int32).reshape(NW * NCH, CH)
    user_rows = _sc_gather(user_table, uidx)
    item_rows = _sc_gather(item_table, iidx)
    user_embedding = _mlp_tc(user_rows, uW1, ub1.reshape(1, H1),
                             uW2, ub2.reshape(1, H2), uW3, ub3.reshape(1, H3))
    item_embedding = _mlp_tc(item_rows, iW1, ib1.reshape(1, H1),
                             iW2, ib2.reshape(1, H2), iW3, ib3.reshape(1, H3))
    return user_embedding, item_embedding
